# baseline (speedup 1.0000x reference)
import functools

import jax
import jax.numpy as jnp
from jax import lax
from jax.experimental import pallas as pl
from jax.experimental.pallas import tpu as pltpu
from jax.experimental.pallas import tpu_sc as plsc

NN = 10000
NP = 10240
EE = 320000
ER = 2500
ERP = 2560
DD = 128
HH = 128
H2 = 64
CC = 10
BN_EPS = 1e-5

NSC = 2
NTILES = 16
CH = 128
NCHUNK = 80
NB = 2
RPT = NP // NTILES

BLK = 1000
GRID = NN // BLK
EBLK = ERP // GRID


_MESH = plsc.VectorSubcoreMesh(core_axis_name="c", subcore_axis_name="s")
_SC_PARAMS = pltpu.CompilerParams(use_tc_tiling_on_sc=False)


def _unpack(pidx, st_s, st_d, j, slot):
  for k in range(CH // 16):
    pk = pidx[j, pl.ds(k * 16, 16)]
    st_s[slot, pl.ds(k * 16, 16)] = pk & 16383
    st_d[slot, pl.ds(k * 16, 16)] = jnp.right_shift(pk, 14)


@functools.partial(
    pl.kernel,
    out_type=jax.ShapeDtypeStruct((NP, 256), jnp.float32),
    mesh=_MESH,
    scratch_types=(
        pltpu.VMEM((NCHUNK, CH), jnp.int32),
        pltpu.VMEM((NB, CH), jnp.int32),
        pltpu.VMEM((NB, CH), jnp.int32),
        pltpu.VMEM((NB, CH, 128), jnp.float32),
        pltpu.VMEM_SHARED((NP, 128), jnp.float32),
        pltpu.SemaphoreType.DMA,
        pltpu.SemaphoreType.DMA,
    ),
    compiler_params=_SC_PARAMS)
def _segsum(tab, pidx_hbm, zrow, out, pidx, st_s, st_d, rows, acc,
            gsem, ssem):
  c = lax.axis_index("c")
  s = lax.axis_index("s")

  pltpu.sync_copy(pidx_hbm.at[c * NTILES + s], pidx)
  pltpu.sync_copy(zrow, acc.at[pl.ds(s * RPT, RPT)])
  plsc.subcore_barrier()

  def group(g, carry):
    base = g * NB
    gds = []
    for b in range(NB):
      _unpack(pidx, st_s, st_d, base + b, b)
      gds.append(pltpu.async_copy(tab.at[st_s.at[b]], rows.at[b], gsem))
    sds = []
    for b in range(NB):
      gds[b].wait()
      sds.append(pltpu.async_copy(rows.at[b], acc.at[st_d.at[b]],
                                  ssem, add=True))
    for d in sds:
      d.wait()
    return carry

  lax.fori_loop(0, NCHUNK // NB, group, 0)
  plsc.subcore_barrier()

  @pl.when(c == 0)
  def _():
    pltpu.sync_copy(acc.at[pl.ds(s * RPT, RPT)],
                    out.at[pl.ds(s * RPT, RPT), pl.ds(0, 128)])

  @pl.when(c == 1)
  def _():
    pltpu.sync_copy(acc.at[pl.ds(s * RPT, RPT)],
                    out.at[pl.ds(s * RPT, RPT), pl.ds(128, 128)])


@functools.partial(
    pl.kernel,
    out_type=jax.ShapeDtypeStruct((NP, 128), jnp.float32),
    mesh=_MESH,
    scratch_types=(
        pltpu.VMEM((NCHUNK, CH), jnp.int32),
        pltpu.VMEM((4, CH), jnp.int32),
        pltpu.VMEM((CH, 16), jnp.float32),
        pltpu.VMEM_SHARED((NP, 16), jnp.float32),
        pltpu.SemaphoreType.DMA,
    ),
    compiler_params=_SC_PARAMS)
def _segcnt(pidx_hbm, zcnt, ones16, out, pidx, st_d, onesb, cacc, osem):
  c = lax.axis_index("c")
  s = lax.axis_index("s")

  pltpu.sync_copy(pidx_hbm.at[c * NTILES + s], pidx)
  pltpu.sync_copy(ones16, onesb)
  pltpu.sync_copy(zcnt, cacc.at[pl.ds(s * RPT, RPT)])
  plsc.subcore_barrier()

  def group(g, carry):
    base = g * 4
    for b in range(4):
      for k in range(CH // 16):
        pk = pidx[base + b, pl.ds(k * 16, 16)]
        st_d[b, pl.ds(k * 16, 16)] = jnp.right_shift(pk, 14)
    sds = [pltpu.async_copy(onesb, cacc.at[st_d.at[b]], osem, add=True)
           for b in range(4)]
    for d in sds:
      d.wait()
    return carry

  lax.fori_loop(0, NCHUNK // 4, group, 0)
  plsc.subcore_barrier()

  @pl.when(c == 0)
  def _():
    pltpu.sync_copy(cacc.at[pl.ds(s * RPT, RPT)],
                    out.at[pl.ds(s * RPT, RPT), pl.ds(0, 16)])

  @pl.when(c == 1)
  def _():
    pltpu.sync_copy(cacc.at[pl.ds(s * RPT, RPT)],
                    out.at[pl.ds(s * RPT, RPT), pl.ds(16, 16)])


def _mm(a, b_t):
  return lax.dot_general(a, b_t, (((1,), (1,)), ((), ())),
                         preferred_element_type=jnp.float32)


def _tc1_body(x_ref, wl_ref, wr_ref, e_ref, q_ref, r_ref, pidx_ref):
  xb = x_ref[...]
  q_ref[...] = _mm(xb, wl_ref[...])
  r_ref[...] = _mm(xb, wr_ref[...])
  i = pl.program_id(0)
  row = jax.lax.broadcasted_iota(jnp.int32, (EBLK, CH), 0) + i * EBLK
  valid = row < ER
  pidx_ref[...] = jnp.where(valid, e_ref[0] | (e_ref[1] << 14), NN << 14)


def _tc1(x, w1l, w1r, e3):
  return pl.pallas_call(
      _tc1_body,
      grid=(GRID,),
      in_specs=[
          pl.BlockSpec((BLK, DD), lambda i: (i, 0)),
          pl.BlockSpec((HH, DD), lambda i: (0, 0)),
          pl.BlockSpec((HH, DD), lambda i: (0, 0)),
          pl.BlockSpec((2, EBLK, CH), lambda i: (0, i, 0)),
      ],
      out_specs=[
          pl.BlockSpec((BLK, HH), lambda i: (i, 0)),
          pl.BlockSpec((BLK, HH), lambda i: (i, 0)),
          pl.BlockSpec((EBLK, CH), lambda i: (i, 0)),
      ],
      out_shape=[
          jax.ShapeDtypeStruct((NN, HH), jnp.float32),
          jax.ShapeDtypeStruct((NN, HH), jnp.float32),
          jax.ShapeDtypeStruct((ERP, CH), jnp.int32),
      ],
  )(x, w1l, w1r, e3)


def _tc2_body(agg_ref, cnt_ref, r1_ref, b1_ref, g1_ref, be1_ref,
              w2l_ref, w2r_ref, p_ref, r2_ref, scl_ref):
  cnt = cnt_ref[:, 0:1] + cnt_ref[:, 16:17]
  scale = 1.0 / jnp.maximum(cnt, 1.0)
  agg = (agg_ref[:, :128] + agg_ref[:, 128:]) * scale
  h = agg + b1_ref[...] + r1_ref[...]
  h = h * (1.0 / jnp.sqrt(1.0 + BN_EPS)) * g1_ref[...] + be1_ref[...]
  h = jnp.maximum(h, 0.0)
  p_ref[...] = jnp.concatenate(
      [_mm(h, w2l_ref[...]), jnp.zeros((BLK, H2), jnp.float32)], axis=1)
  r2_ref[...] = _mm(h, w2r_ref[...])
  scl_ref[...] = jnp.broadcast_to(scale, (BLK, 16))


def _tc2(agg1, cnt, r1, b1, g1, be1, w2l, w2r):
  return pl.pallas_call(
      _tc2_body,
      grid=(GRID,),
      in_specs=[
          pl.BlockSpec((BLK, 256), lambda i: (i, 0)),
          pl.BlockSpec((BLK, 128), lambda i: (i, 0)),
          pl.BlockSpec((BLK, HH), lambda i: (i, 0)),
          pl.BlockSpec((1, HH), lambda i: (0, 0)),
          pl.BlockSpec((1, HH), lambda i: (0, 0)),
          pl.BlockSpec((1, HH), lambda i: (0, 0)),
          pl.BlockSpec((H2, HH), lambda i: (0, 0)),
          pl.BlockSpec((H2, HH), lambda i: (0, 0)),
      ],
      out_specs=[
          pl.BlockSpec((BLK, 128), lambda i: (i, 0)),
          pl.BlockSpec((BLK, H2), lambda i: (i, 0)),
          pl.BlockSpec((BLK, 16), lambda i: (i, 0)),
      ],
      out_shape=[
          jax.ShapeDtypeStruct((NN, 128), jnp.float32),
          jax.ShapeDtypeStruct((NN, H2), jnp.float32),
          jax.ShapeDtypeStruct((NN, 16), jnp.float32),
      ],
  )(agg1, cnt, r1, b1, g1, be1, w2l, w2r)


def _tc3_body(agg_ref, scl_ref, r2_ref, b2_ref, g2_ref, be2_ref,
              wh_ref, bh_ref, o_ref):
  scale = scl_ref[:, 0:1]
  agg = (agg_ref[:, :H2] + agg_ref[:, 128:128 + H2]) * scale
  h = agg + b2_ref[...] + r2_ref[...]
  h = h * (1.0 / jnp.sqrt(1.0 + BN_EPS)) * g2_ref[...] + be2_ref[...]
  h = jnp.maximum(h, 0.0)
  o_ref[...] = lax.dot_general(h, wh_ref[...], (((1,), (0,)), ((), ())),
                               preferred_element_type=jnp.float32) + bh_ref[...]


def _tc3(agg2, scl, r2, b2, g2, be2, whp, bhp):
  return pl.pallas_call(
      _tc3_body,
      grid=(GRID,),
      in_specs=[
          pl.BlockSpec((BLK, 256), lambda i: (i, 0)),
          pl.BlockSpec((BLK, 16), lambda i: (i, 0)),
          pl.BlockSpec((BLK, H2), lambda i: (i, 0)),
          pl.BlockSpec((1, H2), lambda i: (0, 0)),
          pl.BlockSpec((1, H2), lambda i: (0, 0)),
          pl.BlockSpec((1, H2), lambda i: (0, 0)),
          pl.BlockSpec((H2, 128), lambda i: (0, 0)),
          pl.BlockSpec((1, 128), lambda i: (0, 0)),
      ],
      out_specs=pl.BlockSpec((BLK, 128), lambda i: (i, 0)),
      out_shape=jax.ShapeDtypeStruct((NN, 128), jnp.float32),
  )(agg2, scl, r2, b2, g2, be2, whp, bhp)


def kernel(x, edge_index, W1_l, b1_l, W1_r, g1, be1,
           W2_l, b2_l, W2_r, g2, be2, Wh, bh):
  e3 = edge_index.reshape(2, ER, CH)

  zrow = jnp.zeros((RPT, 128), jnp.float32)
  zcnt = jnp.zeros((RPT, 16), jnp.float32)
  ones16 = jnp.zeros((CH, 16), jnp.float32).at[:, 0].set(1.0)

  q, r1, pidx_f = _tc1(x, W1_l, W1_r, e3)
  pidx = pidx_f.reshape(NSC * NTILES, NCHUNK, CH)
  cnt = _segcnt(pidx, zcnt, ones16)
  agg1 = _segsum(q, pidx, zrow)

  p, r2, scl = _tc2(agg1, cnt, r1, b1_l.reshape(1, HH), g1.reshape(1, HH),
                    be1.reshape(1, HH), W2_l, W2_r)
  agg2 = _segsum(p, pidx, zrow)

  whp = jnp.zeros((H2, 128), jnp.float32).at[:, :CC].set(Wh.T)
  bhp = jnp.zeros((1, 128), jnp.float32).at[0, :CC].set(bh)
  out = _tc3(agg2, scl, r2, b2_l.reshape(1, H2), g2.reshape(1, H2),
             be2.reshape(1, H2), whp, bhp)
  return out[:, :CC]


# speedup vs baseline: 1.0028x; 1.0028x over previous
import functools

import jax
import jax.numpy as jnp
from jax import lax
from jax.experimental import pallas as pl
from jax.experimental.pallas import tpu as pltpu
from jax.experimental.pallas import tpu_sc as plsc

NN = 10000
NP = 10240
EE = 320000
ER = 2500
ERP = 2560
DD = 128
HH = 128
H2 = 64
CC = 10
BN_EPS = 1e-5

NSC = 2
NTILES = 16
CH = 128
NCHUNK = 80
NB = 2
RPT = NP // NTILES

BLK = 1000
GRID = NN // BLK
EBLK = ERP // GRID


_MESH = plsc.VectorSubcoreMesh(core_axis_name="c", subcore_axis_name="s")
_SC_PARAMS = pltpu.CompilerParams(use_tc_tiling_on_sc=False)


def _unpack(pidx, st_s, st_d, j, slot):
  for k in range(CH // 16):
    pk = pidx[j, pl.ds(k * 16, 16)]
    st_s[slot, pl.ds(k * 16, 16)] = pk & 16383
    st_d[slot, pl.ds(k * 16, 16)] = jnp.right_shift(pk, 14)


@functools.partial(
    pl.kernel,
    out_type=jax.ShapeDtypeStruct((NP, 256), jnp.float32),
    mesh=_MESH,
    scratch_types=(
        pltpu.VMEM((NCHUNK, CH), jnp.int32),
        pltpu.VMEM((NB, CH), jnp.int32),
        pltpu.VMEM((NB, CH), jnp.int32),
        pltpu.VMEM((NB, CH, 128), jnp.float32),
        pltpu.VMEM_SHARED((NP, 128), jnp.float32),
        pltpu.SemaphoreType.DMA,
        pltpu.SemaphoreType.DMA,
    ),
    compiler_params=_SC_PARAMS)
def _segsum(tab, pidx_hbm, zrow, out, pidx, st_s, st_d, rows, acc,
            gsem, ssem):
  c = lax.axis_index("c")
  s = lax.axis_index("s")

  pltpu.sync_copy(pidx_hbm.at[c * NTILES + s], pidx)
  pltpu.sync_copy(zrow, acc.at[pl.ds(s * RPT, RPT)])
  plsc.subcore_barrier()

  def group(g, carry):
    base = g * NB
    gds = []
    for b in range(NB):
      _unpack(pidx, st_s, st_d, base + b, b)
      gds.append(pltpu.async_copy(tab.at[st_s.at[b]], rows.at[b], gsem))
    sds = []
    for b in range(NB):
      gds[b].wait()
      sds.append(pltpu.async_copy(rows.at[b], acc.at[st_d.at[b]],
                                  ssem, add=True))
    for d in sds:
      d.wait()
    return carry

  lax.fori_loop(0, NCHUNK // NB, group, 0)
  plsc.subcore_barrier()

  @pl.when(c == 0)
  def _():
    pltpu.sync_copy(acc.at[pl.ds(s * RPT, RPT)],
                    out.at[pl.ds(s * RPT, RPT), pl.ds(0, 128)])

  @pl.when(c == 1)
  def _():
    pltpu.sync_copy(acc.at[pl.ds(s * RPT, RPT)],
                    out.at[pl.ds(s * RPT, RPT), pl.ds(128, 128)])


@functools.partial(
    pl.kernel,
    out_type=jax.ShapeDtypeStruct((NP, 128), jnp.float32),
    mesh=_MESH,
    scratch_types=(
        pltpu.VMEM((NCHUNK, CH), jnp.int32),
        pltpu.VMEM((4, CH), jnp.int32),
        pltpu.VMEM((CH, 16), jnp.float32),
        pltpu.VMEM_SHARED((NP, 16), jnp.float32),
        pltpu.SemaphoreType.DMA,
    ),
    compiler_params=_SC_PARAMS)
def _segcnt(pidx_hbm, zcnt, ones16, out, pidx, st_d, onesb, cacc, osem):
  c = lax.axis_index("c")
  s = lax.axis_index("s")

  pltpu.sync_copy(pidx_hbm.at[c * NTILES + s], pidx)
  pltpu.sync_copy(ones16, onesb)
  pltpu.sync_copy(zcnt, cacc.at[pl.ds(s * RPT, RPT)])
  plsc.subcore_barrier()

  def group(g, carry):
    base = g * 4
    for b in range(4):
      for k in range(CH // 16):
        pk = pidx[base + b, pl.ds(k * 16, 16)]
        st_d[b, pl.ds(k * 16, 16)] = jnp.right_shift(pk, 14)
    sds = [pltpu.async_copy(onesb, cacc.at[st_d.at[b]], osem, add=True)
           for b in range(4)]
    for d in sds:
      d.wait()
    return carry

  lax.fori_loop(0, NCHUNK // 4, group, 0)
  plsc.subcore_barrier()

  @pl.when(c == 0)
  def _():
    pltpu.sync_copy(cacc.at[pl.ds(s * RPT, RPT)],
                    out.at[pl.ds(s * RPT, RPT), pl.ds(0, 16)])

  @pl.when(c == 1)
  def _():
    pltpu.sync_copy(cacc.at[pl.ds(s * RPT, RPT)],
                    out.at[pl.ds(s * RPT, RPT), pl.ds(16, 16)])


def _mm(a, b_t):
  return lax.dot_general(a, b_t, (((1,), (1,)), ((), ())),
                         preferred_element_type=jnp.float32)


def _tc1_body(x_ref, wl_ref, wr_ref, e_ref, q_ref, r_ref, pidx_ref):
  xb = x_ref[...]
  q_ref[...] = _mm(xb, wl_ref[...])
  r_ref[...] = _mm(xb, wr_ref[...])
  i = pl.program_id(0)
  row = jax.lax.broadcasted_iota(jnp.int32, (EBLK, CH), 0) + i * EBLK
  lane = jax.lax.broadcasted_iota(jnp.int32, (EBLK, CH), 1)
  valid = row < ER
  pidx_ref[...] = jnp.where(valid, e_ref[0] | (e_ref[1] << 14),
                            (NN + lane) << 14)


def _tc1(x, w1l, w1r, e3):
  return pl.pallas_call(
      _tc1_body,
      grid=(GRID,),
      in_specs=[
          pl.BlockSpec((BLK, DD), lambda i: (i, 0)),
          pl.BlockSpec((HH, DD), lambda i: (0, 0)),
          pl.BlockSpec((HH, DD), lambda i: (0, 0)),
          pl.BlockSpec((2, EBLK, CH), lambda i: (0, i, 0)),
      ],
      out_specs=[
          pl.BlockSpec((BLK, HH), lambda i: (i, 0)),
          pl.BlockSpec((BLK, HH), lambda i: (i, 0)),
          pl.BlockSpec((EBLK, CH), lambda i: (i, 0)),
      ],
      out_shape=[
          jax.ShapeDtypeStruct((NN, HH), jnp.float32),
          jax.ShapeDtypeStruct((NN, HH), jnp.float32),
          jax.ShapeDtypeStruct((ERP, CH), jnp.int32),
      ],
  )(x, w1l, w1r, e3)


def _tc2_body(agg_ref, cnt_ref, r1_ref, b1_ref, g1_ref, be1_ref,
              w2l_ref, w2r_ref, p_ref, r2_ref, scl_ref):
  cnt = cnt_ref[:, 0:1] + cnt_ref[:, 16:17]
  scale = 1.0 / jnp.maximum(cnt, 1.0)
  agg = (agg_ref[:, :128] + agg_ref[:, 128:]) * scale
  h = agg + b1_ref[...] + r1_ref[...]
  h = h * (1.0 / jnp.sqrt(1.0 + BN_EPS)) * g1_ref[...] + be1_ref[...]
  h = jnp.maximum(h, 0.0)
  p_ref[...] = jnp.concatenate(
      [_mm(h, w2l_ref[...]), jnp.zeros((BLK, H2), jnp.float32)], axis=1)
  r2_ref[...] = _mm(h, w2r_ref[...])
  scl_ref[...] = jnp.broadcast_to(scale, (BLK, 16))


def _tc2(agg1, cnt, r1, b1, g1, be1, w2l, w2r):
  return pl.pallas_call(
      _tc2_body,
      grid=(GRID,),
      in_specs=[
          pl.BlockSpec((BLK, 256), lambda i: (i, 0)),
          pl.BlockSpec((BLK, 128), lambda i: (i, 0)),
          pl.BlockSpec((BLK, HH), lambda i: (i, 0)),
          pl.BlockSpec((1, HH), lambda i: (0, 0)),
          pl.BlockSpec((1, HH), lambda i: (0, 0)),
          pl.BlockSpec((1, HH), lambda i: (0, 0)),
          pl.BlockSpec((H2, HH), lambda i: (0, 0)),
          pl.BlockSpec((H2, HH), lambda i: (0, 0)),
      ],
      out_specs=[
          pl.BlockSpec((BLK, 128), lambda i: (i, 0)),
          pl.BlockSpec((BLK, H2), lambda i: (i, 0)),
          pl.BlockSpec((BLK, 16), lambda i: (i, 0)),
      ],
      out_shape=[
          jax.ShapeDtypeStruct((NN, 128), jnp.float32),
          jax.ShapeDtypeStruct((NN, H2), jnp.float32),
          jax.ShapeDtypeStruct((NN, 16), jnp.float32),
      ],
  )(agg1, cnt, r1, b1, g1, be1, w2l, w2r)


def _tc3_body(agg_ref, scl_ref, r2_ref, b2_ref, g2_ref, be2_ref,
              wh_ref, bh_ref, o_ref):
  scale = scl_ref[:, 0:1]
  agg = (agg_ref[:, :H2] + agg_ref[:, 128:128 + H2]) * scale
  h = agg + b2_ref[...] + r2_ref[...]
  h = h * (1.0 / jnp.sqrt(1.0 + BN_EPS)) * g2_ref[...] + be2_ref[...]
  h = jnp.maximum(h, 0.0)
  o_ref[...] = lax.dot_general(h, wh_ref[...], (((1,), (0,)), ((), ())),
                               preferred_element_type=jnp.float32) + bh_ref[...]


def _tc3(agg2, scl, r2, b2, g2, be2, whp, bhp):
  return pl.pallas_call(
      _tc3_body,
      grid=(GRID,),
      in_specs=[
          pl.BlockSpec((BLK, 256), lambda i: (i, 0)),
          pl.BlockSpec((BLK, 16), lambda i: (i, 0)),
          pl.BlockSpec((BLK, H2), lambda i: (i, 0)),
          pl.BlockSpec((1, H2), lambda i: (0, 0)),
          pl.BlockSpec((1, H2), lambda i: (0, 0)),
          pl.BlockSpec((1, H2), lambda i: (0, 0)),
          pl.BlockSpec((H2, 128), lambda i: (0, 0)),
          pl.BlockSpec((1, 128), lambda i: (0, 0)),
      ],
      out_specs=pl.BlockSpec((BLK, 128), lambda i: (i, 0)),
      out_shape=jax.ShapeDtypeStruct((NN, 128), jnp.float32),
  )(agg2, scl, r2, b2, g2, be2, whp, bhp)


def kernel(x, edge_index, W1_l, b1_l, W1_r, g1, be1,
           W2_l, b2_l, W2_r, g2, be2, Wh, bh):
  e3 = edge_index.reshape(2, ER, CH)

  zrow = jnp.zeros((RPT, 128), jnp.float32)
  zcnt = jnp.zeros((RPT, 16), jnp.float32)
  ones16 = jnp.zeros((CH, 16), jnp.float32).at[:, 0].set(1.0)

  q, r1, pidx_f = _tc1(x, W1_l, W1_r, e3)
  pidx = pidx_f.reshape(NSC * NTILES, NCHUNK, CH)
  cnt = _segcnt(pidx, zcnt, ones16)
  agg1 = _segsum(q, pidx, zrow)

  p, r2, scl = _tc2(agg1, cnt, r1, b1_l.reshape(1, HH), g1.reshape(1, HH),
                    be1.reshape(1, HH), W2_l, W2_r)
  agg2 = _segsum(p, pidx, zrow)

  whp = jnp.zeros((H2, 128), jnp.float32).at[:, :CC].set(Wh.T)
  bhp = jnp.zeros((1, 128), jnp.float32).at[0, :CC].set(bh)
  out = _tc3(agg2, scl, r2, b2_l.reshape(1, H2), g2.reshape(1, H2),
             be2.reshape(1, H2), whp, bhp)
  return out[:, :CC]


# speedup vs baseline: 1.9011x; 1.8957x over previous
import functools

import jax
import jax.numpy as jnp
from jax import lax
from jax.experimental import pallas as pl
from jax.experimental.pallas import tpu as pltpu
from jax.experimental.pallas import tpu_sc as plsc

NN = 10000
NP = 10240
EE = 320000
ER = 2500
ERP = 2560
DD = 128
HH = 128
H2 = 64
CC = 10
BN_EPS = 1e-5

NSC = 2
NTILES = 16
CH = 128
NCHUNK = 160
RPT = NP // NTILES

BLK = 1000
GRID = NN // BLK
EBLK = ERP // GRID


_MESH = plsc.VectorSubcoreMesh(core_axis_name="c", subcore_axis_name="s")
_SC_PARAMS = pltpu.CompilerParams(use_tc_tiling_on_sc=False)


def _make_segsum(width, NB):
  scratch = [
      pltpu.VMEM((NCHUNK, CH), jnp.int32),
      pltpu.VMEM((NCHUNK, CH), jnp.int32),
      pltpu.VMEM((NB, CH, width), jnp.float32),
      pltpu.VMEM_SHARED((NP, width), jnp.float32),
      pltpu.SemaphoreType.DMA,
      pltpu.SemaphoreType.DMA,
  ]

  def body(ta, tb, src_r, dst_r, zrow, out, idx_s, idx_d, rows, acc,
           gsem, ssem):
    c = lax.axis_index("c")
    s = lax.axis_index("s")

    pltpu.sync_copy(src_r.at[s], idx_s)
    pltpu.sync_copy(dst_r.at[s], idx_d)
    pltpu.sync_copy(zrow, acc.at[pl.ds(s * RPT, RPT)])

    plsc.subcore_barrier()

    def run(table):
      def group(g, carry):
        base = g * NB
        gds = [pltpu.async_copy(table.at[idx_s.at[base + b]], rows.at[b], gsem)
               for b in range(NB)]
        sds = []
        for b in range(NB):
          gds[b].wait()
          sds.append(pltpu.async_copy(rows.at[b], acc.at[idx_d.at[base + b]],
                                      ssem, add=True))
        for d in sds:
          d.wait()
        return carry
      lax.fori_loop(0, NCHUNK // NB, group, 0)

    @pl.when(c == 0)
    def _():
      run(ta)

    @pl.when(c == 1)
    def _():
      run(tb)

    plsc.subcore_barrier()

    @pl.when(c == 0)
    def _():
      pltpu.sync_copy(acc.at[pl.ds(s * RPT, RPT)],
                      out.at[pl.ds(s * RPT, RPT), pl.ds(0, width)])

    @pl.when(c == 1)
    def _():
      pltpu.sync_copy(acc.at[pl.ds(s * RPT, RPT)],
                      out.at[pl.ds(s * RPT, RPT), pl.ds(width, width)])

  return functools.partial(
      pl.kernel, out_type=jax.ShapeDtypeStruct((NP, 128), jnp.float32),
      mesh=_MESH, scratch_types=tuple(scratch),
      compiler_params=_SC_PARAMS)(body)


_segsum64 = _make_segsum(H2, 5)
_segsum32 = _make_segsum(H2 // 2, 8)

ECNT = ERP // (NSC * NTILES)


@functools.partial(
    pl.kernel,
    out_type=jax.ShapeDtypeStruct((NP, 128), jnp.float32),
    mesh=_MESH,
    scratch_types=(
        pltpu.VMEM((ECNT, CH), jnp.int32),
        pltpu.VMEM((CH, 16), jnp.float32),
        pltpu.VMEM_SHARED((NP, 16), jnp.float32),
        pltpu.SemaphoreType.DMA,
    ),
    compiler_params=_SC_PARAMS)
def _segcnt(dst_hbm, zcnt, ones16, out, idx_d, onesb, cacc, osem):
  c = lax.axis_index("c")
  s = lax.axis_index("s")
  w = c * NTILES + s

  pltpu.sync_copy(dst_hbm.at[pl.ds(w * ECNT, ECNT)], idx_d)
  pltpu.sync_copy(ones16, onesb)
  pltpu.sync_copy(zcnt, cacc.at[pl.ds(s * RPT, RPT)])
  plsc.subcore_barrier()

  def group(g, carry):
    base = g * 4
    sds = [pltpu.async_copy(onesb, cacc.at[idx_d.at[base + b]], osem,
                            add=True)
           for b in range(4)]
    for d in sds:
      d.wait()
    return carry

  lax.fori_loop(0, ECNT // 4, group, 0)
  plsc.subcore_barrier()

  @pl.when(c == 0)
  def _():
    pltpu.sync_copy(cacc.at[pl.ds(s * RPT, RPT)],
                    out.at[pl.ds(s * RPT, RPT), pl.ds(0, 16)])

  @pl.when(c == 1)
  def _():
    pltpu.sync_copy(cacc.at[pl.ds(s * RPT, RPT)],
                    out.at[pl.ds(s * RPT, RPT), pl.ds(16, 16)])


def _mm(a, b_t):
  return lax.dot_general(a, b_t, (((1,), (1,)), ((), ())),
                         preferred_element_type=jnp.float32)


def _tc1_body(x_ref, wl_ref, wr_ref, e_ref,
              qa_ref, qb_ref, r_ref, src_ref, dst_ref):
  xb = x_ref[...]
  q = _mm(xb, wl_ref[...])
  r_ref[...] = _mm(xb, wr_ref[...])
  qa_ref[...] = q[:, :H2]
  qb_ref[...] = q[:, H2:]
  i = pl.program_id(0)
  row = jax.lax.broadcasted_iota(jnp.int32, (EBLK, CH), 0) + i * EBLK
  lane = jax.lax.broadcasted_iota(jnp.int32, (EBLK, CH), 1)
  valid = row < ER
  src_ref[...] = jnp.where(valid, e_ref[0], 0)
  dst_ref[...] = jnp.where(valid, e_ref[1], NN + lane)


def _tc1(x, w1l, w1r, e3):
  return pl.pallas_call(
      _tc1_body,
      grid=(GRID,),
      in_specs=[
          pl.BlockSpec((BLK, DD), lambda i: (i, 0)),
          pl.BlockSpec((HH, DD), lambda i: (0, 0)),
          pl.BlockSpec((HH, DD), lambda i: (0, 0)),
          pl.BlockSpec((2, EBLK, CH), lambda i: (0, i, 0)),
      ],
      out_specs=[
          pl.BlockSpec((BLK, H2), lambda i: (i, 0)),
          pl.BlockSpec((BLK, H2), lambda i: (i, 0)),
          pl.BlockSpec((BLK, HH), lambda i: (i, 0)),
          pl.BlockSpec((EBLK, CH), lambda i: (i, 0)),
          pl.BlockSpec((EBLK, CH), lambda i: (i, 0)),
      ],
      out_shape=[
          jax.ShapeDtypeStruct((NN, H2), jnp.float32),
          jax.ShapeDtypeStruct((NN, H2), jnp.float32),
          jax.ShapeDtypeStruct((NN, HH), jnp.float32),
          jax.ShapeDtypeStruct((ERP, CH), jnp.int32),
          jax.ShapeDtypeStruct((ERP, CH), jnp.int32),
      ],
  )(x, w1l, w1r, e3)


def _tc2_body(agg_ref, cnt_ref, r1_ref, b1_ref, g1_ref, be1_ref,
              w2l_ref, w2r_ref, pa_ref, pb_ref, r2_ref, scl_ref):
  scale = 1.0 / jnp.maximum(cnt_ref[:, 0:1] + cnt_ref[:, 16:17], 1.0)
  agg = agg_ref[...] * scale
  h = agg + b1_ref[...] + r1_ref[...]
  h = h * (1.0 / jnp.sqrt(1.0 + BN_EPS)) * g1_ref[...] + be1_ref[...]
  h = jnp.maximum(h, 0.0)
  p = _mm(h, w2l_ref[...])
  pa_ref[...] = p[:, :H2 // 2]
  pb_ref[...] = p[:, H2 // 2:]
  r2_ref[...] = _mm(h, w2r_ref[...])
  scl_ref[...] = jnp.broadcast_to(scale, (BLK, 16))


def _tc2(agg1, cnt, r1, b1, g1, be1, w2l, w2r):
  return pl.pallas_call(
      _tc2_body,
      grid=(GRID,),
      in_specs=[
          pl.BlockSpec((BLK, 128), lambda i: (i, 0)),
          pl.BlockSpec((BLK, 128), lambda i: (i, 0)),
          pl.BlockSpec((BLK, HH), lambda i: (i, 0)),
          pl.BlockSpec((1, HH), lambda i: (0, 0)),
          pl.BlockSpec((1, HH), lambda i: (0, 0)),
          pl.BlockSpec((1, HH), lambda i: (0, 0)),
          pl.BlockSpec((H2, HH), lambda i: (0, 0)),
          pl.BlockSpec((H2, HH), lambda i: (0, 0)),
      ],
      out_specs=[
          pl.BlockSpec((BLK, H2 // 2), lambda i: (i, 0)),
          pl.BlockSpec((BLK, H2 // 2), lambda i: (i, 0)),
          pl.BlockSpec((BLK, H2), lambda i: (i, 0)),
          pl.BlockSpec((BLK, 16), lambda i: (i, 0)),
      ],
      out_shape=[
          jax.ShapeDtypeStruct((NN, H2 // 2), jnp.float32),
          jax.ShapeDtypeStruct((NN, H2 // 2), jnp.float32),
          jax.ShapeDtypeStruct((NN, H2), jnp.float32),
          jax.ShapeDtypeStruct((NN, 16), jnp.float32),
      ],
  )(agg1, cnt, r1, b1, g1, be1, w2l, w2r)


def _tc3_body(agg_ref, scl_ref, r2_ref, b2_ref, g2_ref, be2_ref,
              wh_ref, bh_ref, o_ref):
  scale = scl_ref[:, 0:1]
  agg = agg_ref[:, :H2] * scale
  h = agg + b2_ref[...] + r2_ref[...]
  h = h * (1.0 / jnp.sqrt(1.0 + BN_EPS)) * g2_ref[...] + be2_ref[...]
  h = jnp.maximum(h, 0.0)
  o_ref[...] = lax.dot_general(h, wh_ref[...], (((1,), (0,)), ((), ())),
                               preferred_element_type=jnp.float32) + bh_ref[...]


def _tc3(agg2, scl, r2, b2, g2, be2, whp, bhp):
  return pl.pallas_call(
      _tc3_body,
      grid=(GRID,),
      in_specs=[
          pl.BlockSpec((BLK, 128), lambda i: (i, 0)),
          pl.BlockSpec((BLK, 16), lambda i: (i, 0)),
          pl.BlockSpec((BLK, H2), lambda i: (i, 0)),
          pl.BlockSpec((1, H2), lambda i: (0, 0)),
          pl.BlockSpec((1, H2), lambda i: (0, 0)),
          pl.BlockSpec((1, H2), lambda i: (0, 0)),
          pl.BlockSpec((H2, 128), lambda i: (0, 0)),
          pl.BlockSpec((1, 128), lambda i: (0, 0)),
      ],
      out_specs=pl.BlockSpec((BLK, 128), lambda i: (i, 0)),
      out_shape=jax.ShapeDtypeStruct((NN, 128), jnp.float32),
  )(agg2, scl, r2, b2, g2, be2, whp, bhp)


def kernel(x, edge_index, W1_l, b1_l, W1_r, g1, be1,
           W2_l, b2_l, W2_r, g2, be2, Wh, bh):
  e3 = edge_index.reshape(2, ER, CH)

  zrow64 = jnp.zeros((RPT, H2), jnp.float32)
  zrow32 = jnp.zeros((RPT, H2 // 2), jnp.float32)
  zcnt = jnp.zeros((RPT, 16), jnp.float32)
  ones16 = jnp.zeros((CH, 16), jnp.float32).at[:, 0].set(1.0)

  qa, qb, r1, src_f, dst_f = _tc1(x, W1_l, W1_r, e3)
  src_r = src_f.reshape(NTILES, NCHUNK, CH)
  dst_r = dst_f.reshape(NTILES, NCHUNK, CH)
  cnt = _segcnt(dst_f, zcnt, ones16)
  agg1 = _segsum64(qa, qb, src_r, dst_r, zrow64)

  pa, pb, r2, scl = _tc2(agg1, cnt, r1, b1_l.reshape(1, HH), g1.reshape(1, HH),
                         be1.reshape(1, HH), W2_l, W2_r)
  agg2 = _segsum32(pa, pb, src_r, dst_r, zrow32)

  whp = jnp.zeros((H2, 128), jnp.float32).at[:, :CC].set(Wh.T)
  bhp = jnp.zeros((1, 128), jnp.float32).at[0, :CC].set(bh)
  out = _tc3(agg2, scl, r2, b2_l.reshape(1, H2), g2.reshape(1, H2),
             be2.reshape(1, H2), whp, bhp)
  return out[:, :CC]
